# baseline (speedup 1.0000x reference)
import jax
import jax.numpy as jnp
from jax.experimental import pallas as pl

_BN = 160


def _body(x_ref, p_ref, wc_ref, bc_ref, wr_ref, br_ref, cls_ref, reg_ref):
    s_inv = 1.0 / 49.0
    xblk = x_ref[...]
    xhi = xblk.astype(jnp.bfloat16)
    xlo = (xblk - xhi.astype(jnp.float32)).astype(jnp.bfloat16)
    p = p_ref[...]
    acc = jnp.dot(xhi, p, preferred_element_type=jnp.float32)
    acc += jnp.dot(xlo, p, preferred_element_type=jnp.float32)
    xm = acc * s_inv
    cls_ref[...] = (
        jnp.dot(xm, wc_ref[...], preferred_element_type=jnp.float32) + bc_ref[...]
    )
    reg_ref[...] = (
        jnp.dot(xm, wr_ref[...], preferred_element_type=jnp.float32) + br_ref[...]
    )


def kernel(x, W_cls, b_cls, W_reg, b_reg):
    n, c, rh, rw = x.shape
    s = rh * rw
    k1 = W_cls.shape[0]
    k2 = W_reg.shape[0]
    x2 = x.reshape(n, c * s)
    pool = jnp.repeat(jnp.eye(c, dtype=jnp.bfloat16), s, axis=0)
    wct = W_cls.T
    wrt = W_reg.T
    bc2 = b_cls.reshape(1, k1)
    br2 = b_reg.reshape(1, k2)
    cls, reg = pl.pallas_call(
        _body,
        grid=(n // _BN,),
        in_specs=[
            pl.BlockSpec((_BN, c * s), lambda i: (i, 0)),
            pl.BlockSpec((c * s, c), lambda i: (0, 0)),
            pl.BlockSpec((c, k1), lambda i: (0, 0)),
            pl.BlockSpec((1, k1), lambda i: (0, 0)),
            pl.BlockSpec((c, k2), lambda i: (0, 0)),
            pl.BlockSpec((1, k2), lambda i: (0, 0)),
        ],
        out_specs=[
            pl.BlockSpec((_BN, k1), lambda i: (i, 0)),
            pl.BlockSpec((_BN, k2), lambda i: (i, 0)),
        ],
        out_shape=[
            jax.ShapeDtypeStruct((n, k1), jnp.float32),
            jax.ShapeDtypeStruct((n, k2), jnp.float32),
        ],
    )(x2, pool, wct, bc2, wrt, br2)
    return (cls, reg)


# speedup vs baseline: 9.3136x; 9.3136x over previous
import jax
import jax.numpy as jnp
from jax import lax
from jax.experimental import pallas as pl

_BN = 160


def _body(x_ref, wc_ref, bc_ref, wr_ref, br_ref, cls_ref, reg_ref):
    s = x_ref.shape[0]
    xm = jnp.sum(x_ref[...], axis=0) * (1.0 / s)
    dn = (((1,), (1,)), ((), ()))
    cls_ref[...] = (
        lax.dot_general(xm, wc_ref[...], dn, preferred_element_type=jnp.float32)
        + bc_ref[...]
    )
    reg_ref[...] = (
        lax.dot_general(xm, wr_ref[...], dn, preferred_element_type=jnp.float32)
        + br_ref[...]
    )


def kernel(x, W_cls, b_cls, W_reg, b_reg):
    n, c, rh, rw = x.shape
    s = rh * rw
    k1 = W_cls.shape[0]
    k2 = W_reg.shape[0]
    x4 = x.transpose(2, 3, 0, 1).reshape(s, n, c)
    bc2 = b_cls.reshape(1, k1)
    br2 = b_reg.reshape(1, k2)
    cls, reg = pl.pallas_call(
        _body,
        grid=(n // _BN,),
        in_specs=[
            pl.BlockSpec((s, _BN, c), lambda i: (0, i, 0)),
            pl.BlockSpec((k1, c), lambda i: (0, 0)),
            pl.BlockSpec((1, k1), lambda i: (0, 0)),
            pl.BlockSpec((k2, c), lambda i: (0, 0)),
            pl.BlockSpec((1, k2), lambda i: (0, 0)),
        ],
        out_specs=[
            pl.BlockSpec((_BN, k1), lambda i: (i, 0)),
            pl.BlockSpec((_BN, k2), lambda i: (i, 0)),
        ],
        out_shape=[
            jax.ShapeDtypeStruct((n, k1), jnp.float32),
            jax.ShapeDtypeStruct((n, k2), jnp.float32),
        ],
    )(x4, W_cls, bc2, W_reg, br2)
    return (cls, reg)
